# baseline (speedup 1.0000x reference)
import functools

import jax
import jax.numpy as jnp
from jax import lax
from jax.experimental import pallas as pl
from jax.experimental.pallas import tpu as pltpu
from jax.experimental.pallas import tpu_sc as plsc

_NC = 2
_NS = 16
_NW = _NC * _NS

_LANES = 16


def _sc_pool(tpair, toff, rpair, roff, trig2, rate2, B, S, T, E, L):
    assert B % _NW == 0
    spw = B // _NW
    nq = 4
    tpq = T // nq
    rpq = tpq * S
    ch = 80
    assert rpq % ch == 0 and ch % 8 == 0
    nch = rpq // ch
    ej = E // _LANES
    E2 = 2 * E
    rchunks = []
    off = 0
    while off < L:
        n = min(128, L - off)
        if L - off > 128:
            n -= n % 8
        rchunks.append((off, n))
        off += n

    mesh = plsc.VectorSubcoreMesh(core_axis_name="c", subcore_axis_name="s")

    @functools.partial(
        pl.kernel,
        out_type=(
            jax.ShapeDtypeStruct((B * T, E), jnp.float32),
            jax.ShapeDtypeStruct((B, E), jnp.float32),
        ),
        mesh=mesh,
        compiler_params=pltpu.CompilerParams(use_tc_tiling_on_sc=False),
        scratch_types=[
            pltpu.VMEM((2, rpq), jnp.int32),
            pltpu.VMEM((2, rpq + 16), jnp.int32),
            pltpu.VMEM((L,), jnp.int32),
            pltpu.VMEM((L + 16,), jnp.int32),
            pltpu.VMEM((2, rpq, E2), jnp.float32),
            pltpu.VMEM((L, E2), jnp.float32),
            pltpu.VMEM((T, E), jnp.float32),
            pltpu.VMEM((1, E), jnp.float32),
            pltpu.SemaphoreType.DMA,
            pltpu.SemaphoreType.DMA,
            pltpu.SemaphoreType.DMA,
        ],
    )
    def pool(tp_hbm, to_hbm, rp_hbm, ro_hbm, tt_hbm, rt_hbm, tout_hbm,
             rout_hbm, idx_v, off_v, ridx_v, roff_v, buf, rbuf, featv,
             ratev, gsem0, gsem1, rsem):
        wid = lax.axis_index("s") * _NC + lax.axis_index("c")
        base_b = wid * spw
        gsems = (gsem0, gsem1)
        rps = T * S

        def fire_quarter(i, q, hb):
            start = (base_b + i) * rps + q * rpq
            pltpu.sync_copy(tp_hbm.at[pl.ds(start, rpq)], idx_v.at[hb])
            pltpu.sync_copy(to_hbm.at[pl.ds(start, rpq)],
                            off_v.at[hb, pl.ds(0, rpq)])
            for k in range(nch):
                pltpu.async_copy(
                    tt_hbm.at[idx_v.at[hb, pl.ds(k * ch, ch)]],
                    buf.at[hb, pl.ds(k * ch, ch)], gsems[hb])

        def wait_quarter(hb):
            pltpu.make_async_copy(
                tt_hbm.at[pl.ds(0, rpq)], buf.at[hb], gsems[hb]).wait()

        def fire_rate(i):
            start = (base_b + i) * L
            pltpu.sync_copy(rp_hbm.at[pl.ds(start, L)], ridx_v)
            pltpu.sync_copy(ro_hbm.at[pl.ds(start, L)],
                            roff_v.at[pl.ds(0, L)])
            for (o, n) in rchunks:
                pltpu.async_copy(
                    rt_hbm.at[ridx_v.at[pl.ds(o, n)]],
                    rbuf.at[pl.ds(o, n)], rsem)

        def wait_rate():
            pltpu.make_async_copy(
                rt_hbm.at[pl.ds(0, L)], rbuf, rsem).wait()

        def accum_quarter(q, hb):
            def tbody(tt, c):
                accs = [jnp.zeros((_LANES,), jnp.float32) for _ in range(ej)]
                for s in range(S):
                    r = tt * S + s
                    po = off_v[hb, pl.ds(r, 16)][0]
                    for j in range(ej):
                        accs[j] = accs[j] + buf[hb, r,
                                                pl.ds(po + j * _LANES,
                                                      _LANES)]
                for j in range(ej):
                    featv[q * tpq + tt, pl.ds(j * _LANES, _LANES)] = (
                        accs[j] * (1.0 / S))
                return c

            lax.fori_loop(0, tpq, tbody, 0)

        def accum_rate():

            def rbody(s, accs):
                po = roff_v[pl.ds(s, 16)][0]
                return tuple(
                    accs[j] + rbuf[s, pl.ds(po + j * _LANES, _LANES)]
                    for j in range(ej))

            raccs = lax.fori_loop(
                0, L, rbody,
                tuple(jnp.zeros((_LANES,), jnp.float32) for _ in range(ej)))
            for j in range(ej):
                ratev[0, pl.ds(j * _LANES, _LANES)] = raccs[j] * (1.0 / L)

        fire_quarter(0, 0, 0)

        def sample_body(i, carry):
            b = base_b + i
            fire_rate(i)
            fire_quarter(i, 1, 1)
            wait_quarter(0)
            accum_quarter(0, 0)
            fire_quarter(i, 2, 0)
            wait_quarter(1)
            accum_quarter(1, 1)
            fire_quarter(i, 3, 1)
            wait_quarter(0)
            accum_quarter(2, 0)
            nxt = jnp.minimum(i + 1, spw - 1)
            fire_quarter(nxt, 0, 0)
            wait_quarter(1)
            accum_quarter(3, 1)
            pltpu.sync_copy(featv, tout_hbm.at[pl.ds(b * T, T)])
            wait_rate()
            accum_rate()
            pltpu.sync_copy(ratev, rout_hbm.at[pl.ds(b, 1)])
            return carry

        lax.fori_loop(0, spw, sample_body, 0)
        wait_quarter(0)

    return pool(tpair, toff, rpair, roff, trig2, rate2)


def _mlp(trig_feat, rate_feat, W1, b1, W2, b2, W3, b3, B, T, E, H, C):
    blk = 256
    assert B % blk == 0

    def body(tf_ref, rf_ref, w1_ref, b1_ref, w2a_ref, w2b_ref, b2_ref,
             w3_ref, b3_ref, o_ref):
        x = tf_ref[...]
        h1 = jnp.dot(x, w1_ref[...], preferred_element_type=jnp.float32)
        h1 = jnp.maximum(h1 + b1_ref[...], 0.0)
        h2 = (jnp.dot(rf_ref[...], w2a_ref[...],
                      preferred_element_type=jnp.float32)
              + jnp.dot(h1, w2b_ref[...], preferred_element_type=jnp.float32))
        h2 = jnp.maximum(h2 + b2_ref[...], 0.0)
        o_ref[...] = (jnp.dot(h2, w3_ref[...],
                              preferred_element_type=jnp.float32)
                      + b3_ref[...])

    grid = (B // blk,)
    full = lambda shape: pl.BlockSpec(shape, lambda i: (0,) * len(shape))
    return pl.pallas_call(
        body,
        grid=grid,
        in_specs=[
            pl.BlockSpec((blk, T * E), lambda i: (i, 0)),
            pl.BlockSpec((blk, E), lambda i: (i, 0)),
            full((T * E, T)),
            full((1, T)),
            full((E, H)),
            full((T, H)),
            full((1, H)),
            full((H, C)),
            full((1, C)),
        ],
        out_specs=pl.BlockSpec((blk, C), lambda i: (i, 0)),
        out_shape=jax.ShapeDtypeStruct((B, C), jnp.float32),
    )(trig_feat, rate_feat, W1, b1.reshape(1, T), W2[:E], W2[E:],
      b2.reshape(1, H), W3, b3.reshape(1, C))


def kernel(usr_trigram, usr_interacted_rates, trigram_table, rate_table,
           W1, b1, W2, b2, W3, b3):
    B, S, T = usr_trigram.shape
    L = usr_interacted_rates.shape[1]
    V, E = trigram_table.shape
    H = b2.shape[0]
    C = b3.shape[0]

    trig2 = trigram_table.reshape(V // 2, 2 * E)
    rate2 = rate_table.reshape(V // 2, 2 * E)
    tidx = usr_trigram.transpose(0, 2, 1).reshape(B * T * S)
    tpair = tidx >> 1
    toff = (tidx & 1) * E
    ridx = usr_interacted_rates.reshape(B * L)
    rpair = ridx >> 1
    roff = (ridx & 1) * E

    trig_feat, rate_feat = _sc_pool(
        tpair, toff, rpair, roff, trig2, rate2, B, S, T, E, L)
    trig_feat = trig_feat.reshape(B, T * E)
    return _mlp(trig_feat, rate_feat, W1, b1, W2, b2, W3, b3, B, T, E, H, C)
